# Initial kernel scaffold; baseline (speedup 1.0000x reference)
import jax
import jax.numpy as jnp
from jax.experimental import pallas as pl


def kernel(x, edge_index, W_mu, b_mu, W_logstd, b_logstd):
    raise NotImplementedError("write your pallas kernel here")


# speedup vs baseline: 33.0436x; 33.0436x over previous
import functools

import jax
import jax.numpy as jnp
from jax import lax
from jax.experimental import pallas as pl
from jax.experimental.pallas import tpu as pltpu
from jax.experimental.pallas import tpu_sc as plsc

N_NODES = 10000
IN_CH = 128
Z_DIM = 64
N_EDGES = 320000

NC = 2
NS = 16
LANES = 16
NT = NC * NS

NPAD = 10240
RPT = NPAD // NT
NP_HIST = 2
NR_HIST = NPAD // NP_HIST
EPT1 = N_EDGES // NS
CHUNK = 128
NCH2 = 80
EPT2 = CHUNK * NCH2
EPAD = EPT2 * NT
ROWS2 = NPAD // NS
XCH = 64


def _phase1_body(x_hbm, dst_hbm, y_hbm, dinv_hbm, deg_hbm,
                 dst_v, hist_v, deg_v, tmp_v, degown_v, dinv_v, xbuf):
    cid = lax.axis_index("c")
    sid = lax.axis_index("s")
    ob = (cid * NS + sid) * RPT

    pltpu.sync_copy(dst_hbm.at[pl.ds(sid * EPT1, EPT1)], dst_v)

    lane = lax.iota(jnp.int32, 16)
    ones = jnp.ones((16,), jnp.float32)

    for p in range(NP_HIST):
        lo = p * NR_HIST

        def zero_body(i, _):
            hist_v[pl.ds(i * 16, 16)] = jnp.zeros((16,), jnp.float32)
            return 0
        lax.fori_loop(0, (LANES * NR_HIST) // 16, zero_body, 0)

        def hist_body(g, _):
            d16 = dst_v[pl.ds(g * 16, 16)]
            rel = d16 - lo
            m = (rel >= 0) & (rel < NR_HIST)
            addr = lane * NR_HIST + jnp.where(m, rel, 0)
            plsc.addupdate_scatter(hist_v, [addr], ones, mask=m)
            return 0
        lax.fori_loop(0, EPT1 // 16, hist_body, 0)

        def red_body(j, _):
            acc = hist_v[pl.ds(j * 16, 16)]
            for l in range(1, LANES):
                acc = acc + hist_v[pl.ds(l * NR_HIST + j * 16, 16)]
            deg_v[pl.ds(lo + j * 16, 16)] = acc
            return 0
        lax.fori_loop(0, NR_HIST // 16, red_body, 0)

    pltpu.sync_copy(deg_v, deg_hbm.at[pl.ds((cid * NS + sid) * NPAD, NPAD)])
    plsc.subcore_barrier()

    def init_one(j, _):
        degown_v[pl.ds(j * 16, 16)] = jnp.ones((16,), jnp.float32)
        return 0
    lax.fori_loop(0, RPT // 16, init_one, 0)

    for l in range(NS):
        pltpu.sync_copy(deg_hbm.at[pl.ds((cid * NS + l) * NPAD + ob, RPT)], tmp_v)

        def acc_body(j, _):
            degown_v[pl.ds(j * 16, 16)] = (degown_v[pl.ds(j * 16, 16)]
                                           + tmp_v[pl.ds(j * 16, 16)])
            return 0
        lax.fori_loop(0, RPT // 16, acc_body, 0)

    def rs_body(j, _):
        d = degown_v[pl.ds(j * 16, 16)]
        i32v = plsc.bitcast(d, jnp.int32)
        seed = jnp.int32(0x5F3759DF) - lax.shift_right_logical(i32v, 1)
        yv = plsc.bitcast(seed, jnp.float32)
        for _ in range(3):
            yv = yv * (1.5 - 0.5 * d * yv * yv)
        dinv_v[pl.ds(j * 16, 16)] = yv
        return 0
    lax.fori_loop(0, RPT // 16, rs_body, 0)

    pltpu.sync_copy(dinv_v, dinv_hbm.at[pl.ds(ob, RPT)])

    for ch in range(RPT // XCH):
        r0 = ob + ch * XCH
        pltpu.sync_copy(x_hbm.at[pl.ds(r0, XCH), :], xbuf)

        def scale_body(g, _):
            s16 = dinv_v[pl.ds(ch * XCH + g * 16, 16)]
            for r16 in range(16):
                r = g * 16 + r16
                s = s16[r16]
                for q in range(IN_CH // 16):
                    xbuf[r, pl.ds(q * 16, 16)] = xbuf[r, pl.ds(q * 16, 16)] * s
            return 0
        lax.fori_loop(0, XCH // 16, scale_body, 0)
        pltpu.sync_copy(xbuf, y_hbm.at[pl.ds(r0, XCH), :])


def _phase2_body(y_hbm, dinv_hbm, srcp_hbm, dstp_hbm, part_hbm,
                 src_v, dst0, dst1, rows0, rows1, dinvbuf, sem0, sem1,
                 agg_shared):
    cid = lax.axis_index("c")
    sid = lax.axis_index("s")
    wid = cid * NS + sid
    rb = sid * ROWS2

    @pl.when(cid == 0)
    def _():
        for chk in range(ROWS2 // CHUNK):
            rr = rb + chk * CHUNK
            pltpu.sync_copy(y_hbm.at[pl.ds(rr, CHUNK), :], rows0)
            pltpu.sync_copy(rows0, agg_shared.at[pl.ds(rr, CHUNK), :])

    @pl.when(cid == 1)
    def _():
        def zr(i, _):
            for q in range(IN_CH // 16):
                rows0[i, pl.ds(q * 16, 16)] = jnp.zeros((16,), jnp.float32)
            return 0
        lax.fori_loop(0, CHUNK, zr, 0)
        for chk in range(ROWS2 // CHUNK):
            rr = rb + chk * CHUNK
            pltpu.sync_copy(rows0, agg_shared.at[pl.ds(rr, CHUNK), :])

    plsc.subcore_barrier()

    pltpu.sync_copy(srcp_hbm.at[wid], src_v)

    pltpu.sync_copy(dstp_hbm.at[wid, 0, :], dst0)
    pltpu.make_async_copy(y_hbm.at[src_v.at[0]], rows0, sem0).start()

    def edge_body(k, _):
        k0 = 2 * k
        pltpu.sync_copy(dstp_hbm.at[wid, k0 + 1, :], dst1)
        pltpu.make_async_copy(y_hbm.at[src_v.at[k0 + 1]], rows1, sem1).start()
        pltpu.make_async_copy(y_hbm.at[src_v.at[k0]], rows0, sem0).wait()
        pltpu.sync_copy(rows0, agg_shared.at[dst0], add=True)

        @pl.when(k < NCH2 // 2 - 1)
        def _():
            pltpu.sync_copy(dstp_hbm.at[wid, k0 + 2, :], dst0)
            pltpu.make_async_copy(y_hbm.at[src_v.at[k0 + 2]], rows0,
                                  sem0).start()
        pltpu.make_async_copy(y_hbm.at[src_v.at[k0 + 1]], rows1, sem1).wait()
        pltpu.sync_copy(rows1, agg_shared.at[dst1], add=True)
        return 0
    lax.fori_loop(0, NCH2 // 2, edge_body, 0)

    plsc.subcore_barrier()

    pltpu.sync_copy(dinv_hbm.at[pl.ds(rb, ROWS2)], dinvbuf)
    for chk in range(ROWS2 // CHUNK):
        rr = rb + chk * CHUNK
        pltpu.sync_copy(agg_shared.at[pl.ds(rr, CHUNK), :], rows0)

        def sc_body(g, _):
            s16 = dinvbuf[pl.ds(chk * CHUNK + g * 16, 16)]
            for r16 in range(16):
                r = g * 16 + r16
                s = s16[r16]
                for q in range(IN_CH // 16):
                    rows0[r, pl.ds(q * 16, 16)] = (rows0[r, pl.ds(q * 16, 16)]
                                                   * s)
            return 0
        lax.fori_loop(0, CHUNK // 16, sc_body, 0)
        pltpu.sync_copy(rows0, part_hbm.at[cid, pl.ds(rr, CHUNK), :])


_sc_mesh = plsc.VectorSubcoreMesh(core_axis_name="c", subcore_axis_name="s")

_phase1 = pl.kernel(
    _phase1_body,
    out_type=(
        jax.ShapeDtypeStruct((NPAD, IN_CH), jnp.float32),
        jax.ShapeDtypeStruct((NPAD,), jnp.float32),
        jax.ShapeDtypeStruct((NC * NS * NPAD,), jnp.float32),
    ),
    mesh=_sc_mesh,
    compiler_params=pltpu.CompilerParams(needs_layout_passes=False),
    scratch_types=[
        pltpu.VMEM((EPT1,), jnp.int32),
        pltpu.VMEM((LANES * NR_HIST,), jnp.float32),
        pltpu.VMEM((NPAD,), jnp.float32),
        pltpu.VMEM((RPT,), jnp.float32),
        pltpu.VMEM((RPT,), jnp.float32),
        pltpu.VMEM((RPT,), jnp.float32),
        pltpu.VMEM((XCH, IN_CH), jnp.float32),
    ],
)

_phase2 = pl.kernel(
    _phase2_body,
    out_type=jax.ShapeDtypeStruct((NC, NPAD, IN_CH), jnp.float32),
    mesh=_sc_mesh,
    compiler_params=pltpu.CompilerParams(needs_layout_passes=False),
    scratch_types=[
        pltpu.VMEM((NCH2, CHUNK), jnp.int32),
        pltpu.VMEM((CHUNK,), jnp.int32),
        pltpu.VMEM((CHUNK,), jnp.int32),
        pltpu.VMEM((CHUNK, IN_CH), jnp.float32),
        pltpu.VMEM((CHUNK, IN_CH), jnp.float32),
        pltpu.VMEM((ROWS2,), jnp.float32),
        pltpu.SemaphoreType.DMA,
        pltpu.SemaphoreType.DMA,
        pltpu.VMEM_SHARED((NPAD, IN_CH), jnp.float32),
    ],
)


def _mm_body(part_ref, wmu_ref, bmu_ref, wls_ref, bls_ref, mu_ref, ls_ref):
    s = part_ref[0] + part_ref[1]
    mu_ref[...] = (jnp.dot(s, wmu_ref[...], preferred_element_type=jnp.float32)
                   + bmu_ref[...])
    ls_ref[...] = (jnp.dot(s, wls_ref[...], preferred_element_type=jnp.float32)
                   + bls_ref[...])


_BM = 1280

_phase3 = pl.pallas_call(
    _mm_body,
    grid=(NPAD // _BM,),
    in_specs=[
        pl.BlockSpec((NC, _BM, IN_CH), lambda i: (0, i, 0)),
        pl.BlockSpec((IN_CH, Z_DIM), lambda i: (0, 0)),
        pl.BlockSpec((1, Z_DIM), lambda i: (0, 0)),
        pl.BlockSpec((IN_CH, Z_DIM), lambda i: (0, 0)),
        pl.BlockSpec((1, Z_DIM), lambda i: (0, 0)),
    ],
    out_specs=[
        pl.BlockSpec((_BM, Z_DIM), lambda i: (i, 0)),
        pl.BlockSpec((_BM, Z_DIM), lambda i: (i, 0)),
    ],
    out_shape=[
        jax.ShapeDtypeStruct((NPAD, Z_DIM), jnp.float32),
        jax.ShapeDtypeStruct((NPAD, Z_DIM), jnp.float32),
    ],
)


def kernel(x, edge_index, W_mu, b_mu, W_logstd, b_logstd):
    x_pad = jnp.zeros((NPAD, IN_CH), jnp.float32).at[:N_NODES].set(x)
    src = edge_index[0]
    dst = edge_index[1]
    n_pad_e = EPAD - N_EDGES
    pad_idx = N_NODES + (jnp.arange(n_pad_e, dtype=jnp.int32)
                         % (NPAD - N_NODES))
    src_p = jnp.concatenate([src, pad_idx]).reshape(NT, NCH2, CHUNK)
    dst_p = jnp.concatenate([dst, pad_idx]).reshape(NT, NCH2, CHUNK)

    y, dinv, _ = _phase1(x_pad, dst)
    part = _phase2(y, dinv, src_p, dst_p)
    mu, ls = _phase3(part, W_mu, b_mu.reshape(1, Z_DIM),
                     W_logstd, b_logstd.reshape(1, Z_DIM))
    return mu[:N_NODES], ls[:N_NODES]
